# baseline (speedup 1.0000x reference)
import jax
import jax.numpy as jnp
from jax import lax
from jax.experimental import pallas as pl
from jax.experimental.pallas import tpu as pltpu
from jax.experimental.pallas import tpu_sc as plsc

NSEG = 4096
ROWS = 32768
FDIM = 128

NT = 16
LANES = 16
FH = FDIM // 2
RT = ROWS // NT
CH = 128
BCH = 512
NB = RT // BCH
KPB = BCH // CH
SEGT = NSEG // NT
CROWS = NSEG // CH


def _make_half(feat_off):
    mesh = plsc.VectorSubcoreMesh(
        core_axis_name="c", subcore_axis_name="s", num_cores=1,
        num_subcores=NT)

    def body(data_hbm, ids_hbm, out_hbm,
             acc_sh, cnt_sh, ids_v, buf_v, zbuf_v, z2_v, cnt2d_v,
             idx32_v, accv, cntv, recips_v, in_sems, sc_sems):
        s = lax.axis_index("s")

        zero16 = jnp.zeros((LANES,), jnp.float32)
        one16 = jnp.ones((LANES,), jnp.float32)
        iota16 = lax.iota(jnp.int32, LANES)

        @pl.loop(0, CH)
        def _(i):
            for k in range(FH // LANES):
                zbuf_v[i, pl.ds(k * LANES, LANES)] = zero16

        @pl.loop(0, CROWS)
        def _(i):
            for k in range(CH // LANES):
                cnt2d_v[i, pl.ds(k * LANES, LANES)] = zero16

        for i in range(2):
            for k in range(CH // LANES):
                z2_v[i, pl.ds(k * LANES, LANES)] = zero16
        idx32_v[pl.ds(0, LANES)] = iota16
        idx32_v[pl.ds(LANES, LANES)] = iota16 + LANES

        pltpu.sync_copy(ids_hbm.at[s], ids_v)
        for k in range(SEGT // CH):
            pltpu.sync_copy(zbuf_v, acc_sh.at[pl.ds(s * SEGT + k * CH, CH)])
        pltpu.sync_copy(z2_v, cnt_sh.at[pl.ds(s * 2, 2)])

        @pl.loop(0, RT // LANES)
        def _(g):
            row = g // (CH // LANES)
            off = (g % (CH // LANES)) * LANES
            ids = ids_v[row, pl.ds(off, LANES)]
            plsc.addupdate_scatter(
                cnt2d_v, [lax.shift_right_logical(ids, 7),
                          lax.bitwise_and(ids, CH - 1)], one16)

        plsc.subcore_barrier()

        def start_in(J, slot):
            row0 = s * RT + J * BCH
            return pltpu.async_copy(
                data_hbm.at[pl.ds(row0, BCH), pl.ds(feat_off, FH)],
                buf_v.at[slot], in_sems.at[slot])

        start_in(0, 0)
        pending = {0: [], 1: []}
        for J in range(NB):
            slot = J % 2
            if J + 1 < NB:
                for d in pending[1 - slot]:
                    d.wait()
                pending[1 - slot] = []
                start_in(J + 1, 1 - slot)
            pltpu.make_async_copy(
                data_hbm.at[pl.ds(s * RT + J * BCH, BCH),
                            pl.ds(feat_off, FH)],
                buf_v.at[slot], in_sems.at[slot]).wait()
            for k in range(KPB):
                pending[slot].append(pltpu.async_copy(
                    buf_v.at[slot, pl.ds(k * CH, CH)],
                    acc_sh.at[ids_v.at[J * KPB + k]], sc_sems.at[slot],
                    add=True))
        for slot in (0, 1):
            for d in pending[slot]:
                d.wait()

        pltpu.sync_copy(cnt2d_v, cnt_sh.at[idx32_v], add=True)

        plsc.subcore_barrier()

        pltpu.sync_copy(acc_sh.at[pl.ds(s * SEGT, SEGT)], accv)
        pltpu.sync_copy(cnt_sh.at[pl.ds(s * 2, 2)], cntv)

        for r in range(SEGT // CH):
            for g in range(CH // LANES):
                cvec = cntv[r, pl.ds(g * LANES, LANES)]
                recips_v[pl.ds(r * CH + g * LANES, LANES)] = (
                    1.0 / jnp.maximum(cvec, one16))

        @pl.loop(0, SEGT)
        def _(row):
            recip = plsc.load_gather(
                recips_v, [jnp.broadcast_to(row, (LANES,))])
            for k in range(FH // LANES):
                sl = (row, pl.ds(k * LANES, LANES))
                accv[sl] = accv[sl] * recip

        pltpu.sync_copy(accv, out_hbm.at[pl.ds(s * SEGT, SEGT)])

    return pl.kernel(
        body,
        out_type=jax.ShapeDtypeStruct((NSEG, FH), jnp.float32),
        mesh=mesh,
        compiler_params=pltpu.CompilerParams(
            use_tc_tiling_on_sc=False, needs_layout_passes=False),
        scratch_types=[
            pltpu.VMEM_SHARED((NSEG, FH), jnp.float32),
            pltpu.VMEM_SHARED((CROWS, CH), jnp.float32),
            pltpu.VMEM((RT // CH, CH), jnp.int32),
            pltpu.VMEM((2, BCH, FH), jnp.float32),
            pltpu.VMEM((CH, FH), jnp.float32),
            pltpu.VMEM((2, CH), jnp.float32),
            pltpu.VMEM((CROWS, CH), jnp.float32),
            pltpu.VMEM((CROWS,), jnp.int32),
            pltpu.VMEM((SEGT, FH), jnp.float32),
            pltpu.VMEM((2, CH), jnp.float32),
            pltpu.VMEM((SEGT,), jnp.float32),
            pltpu.SemaphoreType.DMA((2,)),
            pltpu.SemaphoreType.DMA((2,)),
        ],
    )


_half_lo = _make_half(0)
_half_hi = _make_half(FH)


@jax.jit
def kernel(data, segment_ids):
    ids3 = segment_ids.reshape(NT, RT // CH, CH)
    lo = _half_lo(data, ids3)
    hi = _half_hi(data, ids3)
    return jnp.concatenate([lo, hi], axis=1)


# speedup vs baseline: 1.7107x; 1.7107x over previous
import jax
import jax.numpy as jnp
from jax import lax
from jax.experimental import pallas as pl
from jax.experimental.pallas import tpu as pltpu
from jax.experimental.pallas import tpu_sc as plsc

NSEG = 4096
ROWS = 32768
FDIM = 128

NC = 2
NT = 16
LANES = 16
FH = FDIM // NC
RT = ROWS // NT
CH = 128
BCH = 512
NB = RT // BCH
KPB = BCH // CH
SEGT = NSEG // NT
CROWS = NSEG // CH

_mesh = plsc.VectorSubcoreMesh(
    core_axis_name="c", subcore_axis_name="s", num_cores=NC, num_subcores=NT
)


def _seg_mean_body(data_hbm, ids_hbm, out_hbm,
                   acc_sh, cnt_sh, ids_v, buf_v, zbuf_v, z2_v, cnt2d_v,
                   idx32_v, accv, cntv, recips_v, in_sems):
    c = lax.axis_index("c")
    s = lax.axis_index("s")

    def start_in(J, slot):
        row0 = s * RT + J * BCH
        return pltpu.async_copy(
            data_hbm.at[pl.ds(row0, BCH), pl.ds(c * FH, FH)],
            buf_v.at[slot], in_sems.at[slot])

    start_in(0, 0)
    start_in(1, 1)

    zero16 = jnp.zeros((LANES,), jnp.float32)
    one16 = jnp.ones((LANES,), jnp.float32)
    iota16 = lax.iota(jnp.int32, LANES)

    @pl.loop(0, CH)
    def _(i):
        for k in range(FH // LANES):
            zbuf_v[i, pl.ds(k * LANES, LANES)] = zero16

    @pl.loop(0, CROWS)
    def _(i):
        for k in range(CH // LANES):
            cnt2d_v[i, pl.ds(k * LANES, LANES)] = zero16

    for i in range(2):
        for k in range(CH // LANES):
            z2_v[i, pl.ds(k * LANES, LANES)] = zero16
    idx32_v[pl.ds(0, LANES)] = iota16
    idx32_v[pl.ds(LANES, LANES)] = iota16 + LANES

    pltpu.sync_copy(ids_hbm.at[s], ids_v)
    for k in range(SEGT // CH):
        pltpu.sync_copy(zbuf_v, acc_sh.at[pl.ds(s * SEGT + k * CH, CH)])
    pltpu.sync_copy(z2_v, cnt_sh.at[pl.ds(s * 2, 2)])

    @pl.loop(0, RT // LANES)
    def _(g):
        row = g // (CH // LANES)
        off = (g % (CH // LANES)) * LANES
        ids = ids_v[row, pl.ds(off, LANES)]
        plsc.addupdate_scatter(
            cnt2d_v, [lax.shift_right_logical(ids, 7),
                      lax.bitwise_and(ids, CH - 1)], one16)

    plsc.subcore_barrier()

    for J in range(NB):
        slot = J % 2
        pltpu.make_async_copy(
            data_hbm.at[pl.ds(s * RT + J * BCH, BCH), pl.ds(c * FH, FH)],
            buf_v.at[slot], in_sems.at[slot]).wait()
        for k in range(KPB):
            pltpu.sync_copy(buf_v.at[slot, pl.ds(k * CH, CH)],
                            acc_sh.at[ids_v.at[J * KPB + k]], add=True)
        if J + 2 < NB:
            start_in(J + 2, slot)

    pltpu.sync_copy(cnt2d_v, cnt_sh.at[idx32_v], add=True)

    plsc.subcore_barrier()

    pltpu.sync_copy(acc_sh.at[pl.ds(s * SEGT, SEGT)], accv)
    pltpu.sync_copy(cnt_sh.at[pl.ds(s * 2, 2)], cntv)

    for r in range(SEGT // CH):
        for g in range(CH // LANES):
            cvec = cntv[r, pl.ds(g * LANES, LANES)]
            recips_v[pl.ds(r * CH + g * LANES, LANES)] = (
                1.0 / jnp.maximum(cvec, one16))

    @pl.loop(0, SEGT)
    def _(row):
        recip = plsc.load_gather(recips_v, [jnp.broadcast_to(row, (LANES,))])
        for k in range(FH // LANES):
            sl = (row, pl.ds(k * LANES, LANES))
            accv[sl] = accv[sl] * recip

    pltpu.sync_copy(accv, out_hbm.at[pl.ds(s * SEGT, SEGT), pl.ds(c * FH, FH)])


@jax.jit
def kernel(data, segment_ids):
    ids3 = segment_ids.reshape(NT, RT // CH, CH)
    seg_mean = pl.kernel(
        _seg_mean_body,
        out_type=jax.ShapeDtypeStruct((NSEG, FDIM), jnp.float32),
        mesh=_mesh,
        compiler_params=pltpu.CompilerParams(
            use_tc_tiling_on_sc=False, needs_layout_passes=False),
        scratch_types=[
            pltpu.VMEM_SHARED((NSEG, FH), jnp.float32),
            pltpu.VMEM_SHARED((CROWS, CH), jnp.float32),
            pltpu.VMEM((RT // CH, CH), jnp.int32),
            pltpu.VMEM((2, BCH, FH), jnp.float32),
            pltpu.VMEM((CH, FH), jnp.float32),
            pltpu.VMEM((2, CH), jnp.float32),
            pltpu.VMEM((CROWS, CH), jnp.float32),
            pltpu.VMEM((CROWS,), jnp.int32),
            pltpu.VMEM((SEGT, FH), jnp.float32),
            pltpu.VMEM((2, CH), jnp.float32),
            pltpu.VMEM((SEGT,), jnp.float32),
            pltpu.SemaphoreType.DMA((2,)),
        ],
    )
    return seg_mean(data, ids3)
